# baseline (speedup 1.0000x reference)
import functools

import jax
import jax.numpy as jnp
from jax import lax
from jax.experimental import pallas as pl
from jax.experimental.pallas import tpu as pltpu
from jax.experimental.pallas import tpu_sc as plsc

B, T, C = 1, 2048, 1024
NH, HD = 16, 64
E, TOPK = 8, 2
BLOCK_M = 128
D_FFN = 2048
NT = T // BLOCK_M
NB = (T * TOPK) // BLOCK_M
R = T * TOPK
_BF = jnp.bfloat16
_F32 = jnp.float32


def _layernorm(xb, w):
    mu = jnp.mean(xb, axis=1, keepdims=True)
    var = jnp.mean((xb - mu) ** 2, axis=1, keepdims=True)
    return (xb - mu) / jnp.sqrt(var + 1e-5) * w


def _k1_body(x_ref, lnw_ref, wattn_ref, qkv_ref):
    h = _layernorm(x_ref[...], lnw_ref[...]).astype(_BF)
    w = wattn_ref[...].astype(_BF)
    qkv_ref[...] = jnp.dot(h, w, preferred_element_type=_F32)


def _k1(x2d, ln1_w2d, w_attn):
    return pl.pallas_call(
        _k1_body,
        grid=(NT,),
        in_specs=[
            pl.BlockSpec((BLOCK_M, C), lambda i: (i, 0)),
            pl.BlockSpec((1, C), lambda i: (0, 0)),
            pl.BlockSpec((C, 3 * C), lambda i: (0, 0)),
        ],
        out_specs=pl.BlockSpec((BLOCK_M, 3 * C), lambda i: (i, 0)),
        out_shape=jax.ShapeDtypeStruct((T, 3 * C), _F32),
    )(x2d, ln1_w2d, w_attn)


BQ = 256
BK = 512
NQ = T // BQ
NKV = T // BK


def _k2_body(q_ref, k_ref, v_ref, o_ref,
             s0_ref, s1_ref, m_ref, l_ref, acc0_ref, acc1_ref,
             kt_ref, vb_ref):
    qb = pl.program_id(1)
    qp = (q_ref[...] * 0.125).astype(_BF)

    @pl.when(qb == 0)
    def _():
        kt_ref[...] = jnp.transpose(k_ref[...].astype(_BF))
        vb_ref[...] = v_ref[...].astype(_BF)

    kt = kt_ref[...]
    vf = vb_ref[...]
    q0, q1 = qp[:, :HD], qp[:, HD:]
    m_ref[...] = jnp.full_like(m_ref, -jnp.inf)
    l_ref[...] = jnp.zeros_like(l_ref)
    acc0_ref[...] = jnp.zeros_like(acc0_ref)
    acc1_ref[...] = jnp.zeros_like(acc1_ref)

    def score(j, masked):
        sl = pl.ds(j * BK, BK)
        s0 = jnp.dot(q0, kt[:HD, j * BK:(j + 1) * BK],
                     preferred_element_type=_F32)
        s1 = jnp.dot(q1, kt[HD:, j * BK:(j + 1) * BK],
                     preferred_element_type=_F32)
        if masked:
            rows = qb * BQ + lax.broadcasted_iota(jnp.int32, (BQ, BK), 0)
            cols = j * BK + lax.broadcasted_iota(jnp.int32, (BQ, BK), 1)
            keep = cols <= rows
            s0 = jnp.where(keep, s0, -jnp.inf)
            s1 = jnp.where(keep, s1, -jnp.inf)
        s0_ref[:, sl] = s0
        s1_ref[:, sl] = s1
        m0 = jnp.max(s0, axis=1, keepdims=True)
        m1 = jnp.max(s1, axis=1, keepdims=True)
        m_ref[...] = jnp.maximum(m_ref[...], jnp.concatenate([m0, m1], axis=1))

    for j in range(NKV):
        executed = j * BK <= qb * BQ + BQ - 1
        below_diag = j * BK + BK - 1 <= qb * BQ

        @pl.when(jnp.logical_and(executed, below_diag))
        def _():
            score(j, masked=False)

        @pl.when(jnp.logical_and(executed, jnp.logical_not(below_diag)))
        def _():
            score(j, masked=True)

    for j in range(NKV):
        @pl.when(j * BK <= qb * BQ + BQ - 1)
        def _():
            sl = pl.ds(j * BK, BK)
            p0 = jnp.exp(s0_ref[:, sl] - m_ref[:, 0:1])
            p1 = jnp.exp(s1_ref[:, sl] - m_ref[:, 1:2])
            l0 = jnp.sum(p0, axis=1, keepdims=True)
            l1 = jnp.sum(p1, axis=1, keepdims=True)
            l_ref[...] += jnp.concatenate([l0, l1], axis=1)
            v0 = vf[j * BK:(j + 1) * BK, :HD]
            v1 = vf[j * BK:(j + 1) * BK, HD:]
            acc0_ref[...] += jnp.dot(p0.astype(_BF), v0,
                                     preferred_element_type=_F32)
            acc1_ref[...] += jnp.dot(p1.astype(_BF), v1,
                                     preferred_element_type=_F32)

    o_ref[...] = jnp.concatenate(
        [acc0_ref[...] / l_ref[:, 0:1], acc1_ref[...] / l_ref[:, 1:2]], axis=1)


def _k2(qkv):
    return pl.pallas_call(
        _k2_body,
        grid=(NH // 2, NQ),
        in_specs=[
            pl.BlockSpec((BQ, 2 * HD), lambda h, i: (i, h)),
            pl.BlockSpec((T, 2 * HD), lambda h, i: (0, 8 + h)),
            pl.BlockSpec((T, 2 * HD), lambda h, i: (0, 16 + h)),
        ],
        out_specs=pl.BlockSpec((BQ, 2 * HD), lambda h, i: (i, h)),
        out_shape=jax.ShapeDtypeStruct((T, C), _F32),
        scratch_shapes=[pltpu.VMEM((BQ, T), _F32),
                        pltpu.VMEM((BQ, T), _F32),
                        pltpu.VMEM((BQ, 2), _F32),
                        pltpu.VMEM((BQ, 2), _F32),
                        pltpu.VMEM((BQ, HD), _F32),
                        pltpu.VMEM((BQ, HD), _F32),
                        pltpu.VMEM((2 * HD, T), _BF),
                        pltpu.VMEM((T, 2 * HD), _BF)],
    )(qkv, qkv, qkv)


def _k3_body(y_ref, x_ref, wp_ref, lnw_ref, wr_ref, x2_ref, h2_ref, lg_ref):
    yb = y_ref[...].astype(_BF)
    wp = wp_ref[...].astype(_BF)
    proj = jnp.dot(yb, wp, preferred_element_type=_F32)
    x2 = x_ref[...] + proj
    x2_ref[...] = x2
    h2 = _layernorm(x2, lnw_ref[...])
    h2_ref[...] = h2
    lg_ref[...] = jnp.dot(h2, wr_ref[...], preferred_element_type=_F32)


def _k3(y2d, x2d, wp, ln2_w2d, w_router):
    return pl.pallas_call(
        _k3_body,
        grid=(NT,),
        in_specs=[
            pl.BlockSpec((BLOCK_M, C), lambda i: (i, 0)),
            pl.BlockSpec((BLOCK_M, C), lambda i: (i, 0)),
            pl.BlockSpec((C, C), lambda i: (0, 0)),
            pl.BlockSpec((1, C), lambda i: (0, 0)),
            pl.BlockSpec((C, E), lambda i: (0, 0)),
        ],
        out_specs=[
            pl.BlockSpec((BLOCK_M, C), lambda i: (i, 0)),
            pl.BlockSpec((BLOCK_M, C), lambda i: (i, 0)),
            pl.BlockSpec((BLOCK_M, E), lambda i: (i, 0)),
        ],
        out_shape=[
            jax.ShapeDtypeStruct((T, C), _F32),
            jax.ShapeDtypeStruct((T, C), _F32),
            jax.ShapeDtypeStruct((T, E), _F32),
        ],
    )(y2d, x2d, wp, ln2_w2d, w_router)


def _k4_body(lg_ref, ranks_ref, rw_ref, col_ref):
    lg = lg_ref[...]
    m = jnp.max(lg, axis=1, keepdims=True)
    ex = jnp.exp(lg - m)
    p = ex / jnp.sum(ex, axis=1, keepdims=True)
    ii = lax.broadcasted_iota(jnp.int32, (T, E), 1)
    v0 = jnp.max(p, axis=1, keepdims=True)
    e0 = jnp.min(jnp.where(p == v0, ii, E), axis=1, keepdims=True)
    p2 = jnp.where(ii == e0, -1.0, p)
    v1 = jnp.max(p2, axis=1, keepdims=True)
    e1 = jnp.min(jnp.where(p2 == v1, ii, E), axis=1, keepdims=True)
    s = v0 + v1
    oh0 = (ii == e0).astype(_F32)
    oh1 = (ii == e1).astype(_F32)
    mass = oh0 + oh1

    ri = lax.broadcasted_iota(jnp.int32, (BLOCK_M, BLOCK_M), 0)
    ci = lax.broadcasted_iota(jnp.int32, (BLOCK_M, BLOCK_M), 1)
    ltri = (ci < ri).astype(_F32)
    carry = jnp.zeros((1, E), _F32)
    parts = []
    for c in range(NT):
        mc = mass[c * BLOCK_M:(c + 1) * BLOCK_M]
        parts.append(jnp.dot(ltri, mc, preferred_element_type=_F32) + carry)
        carry = carry + jnp.sum(mc, axis=0, keepdims=True)
    pref = jnp.concatenate(parts, axis=0)
    ur = lax.broadcasted_iota(jnp.int32, (E, E), 0)
    uc = lax.broadcasted_iota(jnp.int32, (E, E), 1)
    offs = jnp.dot(carry, (ur < uc).astype(_F32),
                   preferred_element_type=_F32,
                   precision=jax.lax.Precision.HIGHEST)
    base = pref + offs
    rank0 = jnp.sum(oh0 * base, axis=1, keepdims=True)
    rank1 = jnp.sum(oh1 * (base + oh0), axis=1, keepdims=True)
    ranks_ref[...] = jnp.concatenate(
        [rank0, rank1], axis=1).astype(jnp.int32)
    rw_ref[...] = jnp.concatenate([v0 / s, v1 / s], axis=1)

    bi = lax.broadcasted_iota(jnp.int32, (NB, E), 0).astype(_F32) * BLOCK_M
    cmp = (jnp.broadcast_to(offs, (NB, E)) <= bi).astype(jnp.int32)
    col = jnp.sum(cmp, axis=1, keepdims=True) - 1
    col_ref[...] = jnp.broadcast_to(col, (NB, E))


def _k4(lg):
    return pl.pallas_call(
        _k4_body,
        out_shape=[
            jax.ShapeDtypeStruct((T, 2), jnp.int32),
            jax.ShapeDtypeStruct((T, 2), _F32),
            jax.ShapeDtypeStruct((NB, E), jnp.int32),
        ],
    )(lg)


@functools.cache
def _sc_mesh():
    return plsc.VectorSubcoreMesh(core_axis_name="c", subcore_axis_name="s")
_NW = 32
_CH = 64


def _sc_scatter(h2, idx_all):
    @functools.partial(
        pl.kernel,
        mesh=_sc_mesh(),
        out_type=jax.ShapeDtypeStruct((R, C), _F32),
        scratch_types=[pltpu.VMEM((_CH,), jnp.int32),
                       pltpu.VMEM((_CH, C), _F32)],
    )
    def k(h2_hbm, idx_hbm, out_hbm, idx_v, rows_v):
        wid = lax.axis_index("s") * 2 + lax.axis_index("c")
        base = wid * (R // _NW)
        for sub in range(R // _NW // _CH):
            off = base + sub * _CH
            src = lax.rem(off, T)
            pltpu.sync_copy(idx_hbm.at[pl.ds(off, _CH)], idx_v)
            pltpu.sync_copy(h2_hbm.at[pl.ds(src, _CH)], rows_v)
            pltpu.sync_copy(rows_v, out_hbm.at[idx_v])

    return k(h2, idx_all)


def _sc_gather(y2, idx_all):
    @functools.partial(
        pl.kernel,
        mesh=_sc_mesh(),
        out_type=jax.ShapeDtypeStruct((R, C), _F32),
        scratch_types=[pltpu.VMEM((_CH,), jnp.int32),
                       pltpu.VMEM((_CH, C), _F32)],
    )
    def k(y2_hbm, idx_hbm, out_hbm, idx_v, rows_v):
        wid = lax.axis_index("s") * 2 + lax.axis_index("c")
        base = wid * (R // _NW)
        for sub in range(R // _NW // _CH):
            off = base + sub * _CH
            pltpu.sync_copy(idx_hbm.at[pl.ds(off, _CH)], idx_v)
            pltpu.sync_copy(y2_hbm.at[idx_v], rows_v)
            pltpu.sync_copy(rows_v, out_hbm.at[pl.ds(off, _CH)])

    return k(y2, idx_all)


def _k6_body(col_ref, xg_ref, w1_ref, w2_ref, o_ref):
    del col_ref
    xb = xg_ref[...].astype(_BF)
    h = jnp.dot(xb, w1_ref[...], preferred_element_type=_F32)
    g = (0.5 * h * (1.0 + lax.erf(h * 0.7071067811865476))).astype(_BF)
    o_ref[...] = jnp.dot(g, w2_ref[...], preferred_element_type=_F32)


def _k6(col, xg, w1, w2):
    grid_spec = pltpu.PrefetchScalarGridSpec(
        num_scalar_prefetch=1,
        grid=(NB,),
        in_specs=[
            pl.BlockSpec((BLOCK_M, C), lambda b, col: (b, 0)),
            pl.BlockSpec((C, D_FFN), lambda b, col: (0, col[b])),
            pl.BlockSpec((D_FFN, C), lambda b, col: (col[b], 0)),
        ],
        out_specs=pl.BlockSpec((BLOCK_M, C), lambda b, col: (b, 0)),
    )
    return pl.pallas_call(
        _k6_body,
        grid_spec=grid_spec,
        out_shape=jax.ShapeDtypeStruct((R, C), _F32),
    )(col, xg, w1, w2)


def _k8_body(x2_ref, g0_ref, g1_ref, rw_ref, o_ref):
    rw = rw_ref[...]
    o_ref[...] = (x2_ref[...] + rw[:, 0:1] * g0_ref[...]
                  + rw[:, 1:2] * g1_ref[...])


def _k8(x2, g, rw):
    return pl.pallas_call(
        _k8_body,
        grid=(NT,),
        in_specs=[
            pl.BlockSpec((BLOCK_M, C), lambda i: (i, 0)),
            pl.BlockSpec((BLOCK_M, C), lambda i: (i, 0)),
            pl.BlockSpec((BLOCK_M, C), lambda i: (i + NT, 0)),
            pl.BlockSpec((BLOCK_M, 2), lambda i: (i, 0)),
        ],
        out_specs=pl.BlockSpec((BLOCK_M, C), lambda i: (i, 0)),
        out_shape=jax.ShapeDtypeStruct((T, C), _F32),
    )(x2, g, g, rw)


def kernel(x, ln1_w, w_attn, w_proj, ln2_w, w_router, w1, w2):
    x2d = x.reshape(T, C)
    qkv = _k1(x2d, ln1_w.reshape(1, C), w_attn)
    y2d = _k2(qkv)
    x2, h2, lg = _k3(y2d, x2d, w_proj, ln2_w.reshape(1, C), w_router)
    ranks, rw, colr = _k4(lg)
    idx_all = ranks.T.reshape(R)
    col = colr[:, 0]
    xg = _sc_scatter(h2, idx_all)
    y2 = _k6(col, xg, w1.astype(_BF), w2.astype(_BF))
    g = _sc_gather(y2, idx_all)
    out = _k8(x2, g, rw)
    return out.reshape(B, T, C), lg


# speedup vs baseline: 1.0885x; 1.0885x over previous
import functools

import jax
import jax.numpy as jnp
from jax import lax
from jax.experimental import pallas as pl
from jax.experimental.pallas import tpu as pltpu
from jax.experimental.pallas import tpu_sc as plsc

B, T, C = 1, 2048, 1024
NH, HD = 16, 64
E, TOPK = 8, 2
BLOCK_M = 128
D_FFN = 2048
NT = T // BLOCK_M
NB = (T * TOPK) // BLOCK_M
R = T * TOPK
_BF = jnp.bfloat16
_F32 = jnp.float32


def _layernorm(xb, w):
    mu = jnp.mean(xb, axis=1, keepdims=True)
    var = jnp.mean((xb - mu) ** 2, axis=1, keepdims=True)
    return (xb - mu) / jnp.sqrt(var + 1e-5) * w


def _k1_body(x_ref, lnw_ref, wattn_ref, qkv_ref):
    h = _layernorm(x_ref[...], lnw_ref[...]).astype(_BF)
    w = wattn_ref[...].astype(_BF)
    qkv_ref[...] = jnp.dot(h, w, preferred_element_type=_F32)


def _k1(x2d, ln1_w2d, w_attn):
    return pl.pallas_call(
        _k1_body,
        grid=(NT,),
        in_specs=[
            pl.BlockSpec((BLOCK_M, C), lambda i: (i, 0)),
            pl.BlockSpec((1, C), lambda i: (0, 0)),
            pl.BlockSpec((C, 3 * C), lambda i: (0, 0)),
        ],
        out_specs=pl.BlockSpec((BLOCK_M, 3 * C), lambda i: (i, 0)),
        out_shape=jax.ShapeDtypeStruct((T, 3 * C), _F32),
    )(x2d, ln1_w2d, w_attn)


BQ = 256
BK = 512
NQ = T // BQ
NKV = T // BK


def _k2_body(q_ref, k_ref, v_ref, o_ref,
             s0_ref, s1_ref, m_ref, l_ref, acc0_ref, acc1_ref,
             kt_ref, vb_ref):
    qb = pl.program_id(1)
    qp = (q_ref[...] * 0.125).astype(_BF)

    @pl.when(qb == 0)
    def _():
        kt_ref[...] = jnp.transpose(k_ref[...].astype(_BF))
        vb_ref[...] = v_ref[...].astype(_BF)

    kt = kt_ref[...]
    vf = vb_ref[...]
    q0, q1 = qp[:, :HD], qp[:, HD:]
    m_ref[...] = jnp.full_like(m_ref, -jnp.inf)
    l_ref[...] = jnp.zeros_like(l_ref)
    acc0_ref[...] = jnp.zeros_like(acc0_ref)
    acc1_ref[...] = jnp.zeros_like(acc1_ref)

    def score(j, masked):
        sl = pl.ds(j * BK, BK)
        s0 = jnp.dot(q0, kt[:HD, j * BK:(j + 1) * BK],
                     preferred_element_type=_F32)
        s1 = jnp.dot(q1, kt[HD:, j * BK:(j + 1) * BK],
                     preferred_element_type=_F32)
        if masked:
            rows = qb * BQ + lax.broadcasted_iota(jnp.int32, (BQ, BK), 0)
            cols = j * BK + lax.broadcasted_iota(jnp.int32, (BQ, BK), 1)
            keep = cols <= rows
            s0 = jnp.where(keep, s0, -jnp.inf)
            s1 = jnp.where(keep, s1, -jnp.inf)
        s0_ref[:, sl] = s0
        s1_ref[:, sl] = s1
        m0 = jnp.max(s0, axis=1, keepdims=True)
        m1 = jnp.max(s1, axis=1, keepdims=True)
        m_ref[...] = jnp.maximum(m_ref[...], jnp.concatenate([m0, m1], axis=1))

    for j in range(NKV):
        executed = j * BK <= qb * BQ + BQ - 1
        below_diag = j * BK + BK - 1 <= qb * BQ

        @pl.when(jnp.logical_and(executed, below_diag))
        def _():
            score(j, masked=False)

        @pl.when(jnp.logical_and(executed, jnp.logical_not(below_diag)))
        def _():
            score(j, masked=True)

    for j in range(NKV):
        @pl.when(j * BK <= qb * BQ + BQ - 1)
        def _():
            sl = pl.ds(j * BK, BK)
            p0 = jnp.exp(s0_ref[:, sl] - m_ref[:, 0:1])
            p1 = jnp.exp(s1_ref[:, sl] - m_ref[:, 1:2])
            l0 = jnp.sum(p0, axis=1, keepdims=True)
            l1 = jnp.sum(p1, axis=1, keepdims=True)
            l_ref[...] += jnp.concatenate([l0, l1], axis=1)
            v0 = vf[j * BK:(j + 1) * BK, :HD]
            v1 = vf[j * BK:(j + 1) * BK, HD:]
            acc0_ref[...] += jnp.dot(p0.astype(_BF), v0,
                                     preferred_element_type=_F32)
            acc1_ref[...] += jnp.dot(p1.astype(_BF), v1,
                                     preferred_element_type=_F32)

    o_ref[...] = jnp.concatenate(
        [acc0_ref[...] / l_ref[:, 0:1], acc1_ref[...] / l_ref[:, 1:2]], axis=1)


def _k2(qkv):
    return pl.pallas_call(
        _k2_body,
        grid=(NH // 2, NQ),
        in_specs=[
            pl.BlockSpec((BQ, 2 * HD), lambda h, i: (i, h)),
            pl.BlockSpec((T, 2 * HD), lambda h, i: (0, 8 + h)),
            pl.BlockSpec((T, 2 * HD), lambda h, i: (0, 16 + h)),
        ],
        out_specs=pl.BlockSpec((BQ, 2 * HD), lambda h, i: (i, h)),
        out_shape=jax.ShapeDtypeStruct((T, C), _F32),
        scratch_shapes=[pltpu.VMEM((BQ, T), _F32),
                        pltpu.VMEM((BQ, T), _F32),
                        pltpu.VMEM((BQ, 2), _F32),
                        pltpu.VMEM((BQ, 2), _F32),
                        pltpu.VMEM((BQ, HD), _F32),
                        pltpu.VMEM((BQ, HD), _F32),
                        pltpu.VMEM((2 * HD, T), _BF),
                        pltpu.VMEM((T, 2 * HD), _BF)],
    )(qkv, qkv, qkv)


def _k3_body(y_ref, x_ref, wp_ref, lnw_ref, wr_ref, x2_ref, h2_ref, lg_ref):
    yb = y_ref[...].astype(_BF)
    wp = wp_ref[...].astype(_BF)
    proj = jnp.dot(yb, wp, preferred_element_type=_F32)
    x2 = x_ref[...] + proj
    x2_ref[...] = x2
    h2 = _layernorm(x2, lnw_ref[...])
    h2_ref[...] = h2
    lg_ref[...] = jnp.dot(h2, wr_ref[...], preferred_element_type=_F32)


def _k3(y2d, x2d, wp, ln2_w2d, w_router):
    return pl.pallas_call(
        _k3_body,
        grid=(NT,),
        in_specs=[
            pl.BlockSpec((BLOCK_M, C), lambda i: (i, 0)),
            pl.BlockSpec((BLOCK_M, C), lambda i: (i, 0)),
            pl.BlockSpec((C, C), lambda i: (0, 0)),
            pl.BlockSpec((1, C), lambda i: (0, 0)),
            pl.BlockSpec((C, E), lambda i: (0, 0)),
        ],
        out_specs=[
            pl.BlockSpec((BLOCK_M, C), lambda i: (i, 0)),
            pl.BlockSpec((BLOCK_M, C), lambda i: (i, 0)),
            pl.BlockSpec((BLOCK_M, E), lambda i: (i, 0)),
        ],
        out_shape=[
            jax.ShapeDtypeStruct((T, C), _F32),
            jax.ShapeDtypeStruct((T, C), _F32),
            jax.ShapeDtypeStruct((T, E), _F32),
        ],
    )(y2d, x2d, wp, ln2_w2d, w_router)


def _k4_body(lg_ref, ranks_ref, rw_ref, col_ref):
    lg = lg_ref[...]
    m = jnp.max(lg, axis=1, keepdims=True)
    ex = jnp.exp(lg - m)
    p = ex / jnp.sum(ex, axis=1, keepdims=True)
    ii = lax.broadcasted_iota(jnp.int32, (T, E), 1)
    v0 = jnp.max(p, axis=1, keepdims=True)
    e0 = jnp.min(jnp.where(p == v0, ii, E), axis=1, keepdims=True)
    p2 = jnp.where(ii == e0, -1.0, p)
    v1 = jnp.max(p2, axis=1, keepdims=True)
    e1 = jnp.min(jnp.where(p2 == v1, ii, E), axis=1, keepdims=True)
    s = v0 + v1
    oh0 = (ii == e0).astype(_F32)
    oh1 = (ii == e1).astype(_F32)
    mass = oh0 + oh1

    ri = lax.broadcasted_iota(jnp.int32, (BLOCK_M, BLOCK_M), 0)
    ci = lax.broadcasted_iota(jnp.int32, (BLOCK_M, BLOCK_M), 1)
    ltri = (ci < ri).astype(_F32)
    carry = jnp.zeros((1, E), _F32)
    parts = []
    for c in range(NT):
        mc = mass[c * BLOCK_M:(c + 1) * BLOCK_M]
        parts.append(jnp.dot(ltri, mc, preferred_element_type=_F32) + carry)
        carry = carry + jnp.sum(mc, axis=0, keepdims=True)
    pref = jnp.concatenate(parts, axis=0)
    ur = lax.broadcasted_iota(jnp.int32, (E, E), 0)
    uc = lax.broadcasted_iota(jnp.int32, (E, E), 1)
    offs = jnp.dot(carry, (ur < uc).astype(_F32),
                   preferred_element_type=_F32,
                   precision=jax.lax.Precision.HIGHEST)
    base = pref + offs
    rank0 = jnp.sum(oh0 * base, axis=1, keepdims=True)
    rank1 = jnp.sum(oh1 * (base + oh0), axis=1, keepdims=True)
    ranks_ref[...] = jnp.concatenate(
        [rank0, rank1], axis=1).astype(jnp.int32)
    rw_ref[...] = jnp.concatenate([v0 / s, v1 / s], axis=1)

    bi = lax.broadcasted_iota(jnp.int32, (NB, E), 0).astype(_F32) * BLOCK_M
    cmp = (jnp.broadcast_to(offs, (NB, E)) <= bi).astype(jnp.int32)
    col = jnp.sum(cmp, axis=1, keepdims=True) - 1
    col_ref[...] = jnp.broadcast_to(col, (NB, E))


def _k4(lg):
    return pl.pallas_call(
        _k4_body,
        out_shape=[
            jax.ShapeDtypeStruct((T, 2), jnp.int32),
            jax.ShapeDtypeStruct((T, 2), _F32),
            jax.ShapeDtypeStruct((NB, E), jnp.int32),
        ],
    )(lg)


@functools.cache
def _sc_mesh():
    return plsc.VectorSubcoreMesh(core_axis_name="c", subcore_axis_name="s")
_NW = 32
_CH = 64


def _sc_scatter(h2, idx_all):
    @functools.partial(
        pl.kernel,
        mesh=_sc_mesh(),
        out_type=jax.ShapeDtypeStruct((R, C), _F32),
        scratch_types=[pltpu.VMEM((_CH,), jnp.int32),
                       pltpu.VMEM((_CH, C), _F32)],
    )
    def k(h2_hbm, idx_hbm, out_hbm, idx_v, rows_v):
        wid = lax.axis_index("s") * 2 + lax.axis_index("c")
        base = wid * (R // _NW)
        for sub in range(R // _NW // _CH):
            off = base + sub * _CH
            src = lax.rem(off, T)
            pltpu.sync_copy(idx_hbm.at[pl.ds(off, _CH)], idx_v)
            pltpu.sync_copy(h2_hbm.at[pl.ds(src, _CH)], rows_v)
            pltpu.sync_copy(rows_v, out_hbm.at[idx_v])

    return k(h2, idx_all)


def _sc_gather(y2, idx_all):
    @functools.partial(
        pl.kernel,
        mesh=_sc_mesh(),
        out_type=jax.ShapeDtypeStruct((R, C), _F32),
        scratch_types=[pltpu.VMEM((_CH,), jnp.int32),
                       pltpu.VMEM((_CH, C), _F32)],
    )
    def k(y2_hbm, idx_hbm, out_hbm, idx_v, rows_v):
        wid = lax.axis_index("s") * 2 + lax.axis_index("c")
        base = wid * (R // _NW)
        for sub in range(R // _NW // _CH):
            off = base + sub * _CH
            pltpu.sync_copy(idx_hbm.at[pl.ds(off, _CH)], idx_v)
            pltpu.sync_copy(y2_hbm.at[idx_v], rows_v)
            pltpu.sync_copy(rows_v, out_hbm.at[pl.ds(off, _CH)])

    return k(y2, idx_all)


def _k6_body(col_ref, xg_ref, w1_ref, w2_ref, o_ref, w1b_ref, w2b_ref):
    b = pl.program_id(0)
    prev = col_ref[jnp.maximum(b - 1, 0)]
    changed = jnp.logical_or(b == 0, col_ref[b] != prev)

    @pl.when(changed)
    def _():
        w1b_ref[...] = w1_ref[...].astype(_BF)
        w2b_ref[...] = w2_ref[...].astype(_BF)

    xb = xg_ref[...].astype(_BF)
    h = jnp.dot(xb, w1b_ref[...], preferred_element_type=_F32)
    g = (0.5 * h * (1.0 + lax.erf(h * 0.7071067811865476))).astype(_BF)
    o_ref[...] = jnp.dot(g, w2b_ref[...], preferred_element_type=_F32)


def _k6(col, xg, w1, w2):
    grid_spec = pltpu.PrefetchScalarGridSpec(
        num_scalar_prefetch=1,
        grid=(NB,),
        in_specs=[
            pl.BlockSpec((BLOCK_M, C), lambda b, col: (b, 0)),
            pl.BlockSpec((C, D_FFN), lambda b, col: (0, col[b])),
            pl.BlockSpec((D_FFN, C), lambda b, col: (col[b], 0)),
        ],
        out_specs=pl.BlockSpec((BLOCK_M, C), lambda b, col: (b, 0)),
        scratch_shapes=[pltpu.VMEM((C, D_FFN), _BF),
                        pltpu.VMEM((D_FFN, C), _BF)],
    )
    return pl.pallas_call(
        _k6_body,
        grid_spec=grid_spec,
        out_shape=jax.ShapeDtypeStruct((R, C), _F32),
    )(col, xg, w1, w2)


def _k8_body(x2_ref, g0_ref, g1_ref, rw_ref, o_ref):
    rw = rw_ref[...]
    o_ref[...] = (x2_ref[...] + rw[:, 0:1] * g0_ref[...]
                  + rw[:, 1:2] * g1_ref[...])


def _k8(x2, g, rw):
    return pl.pallas_call(
        _k8_body,
        grid=(NT,),
        in_specs=[
            pl.BlockSpec((BLOCK_M, C), lambda i: (i, 0)),
            pl.BlockSpec((BLOCK_M, C), lambda i: (i, 0)),
            pl.BlockSpec((BLOCK_M, C), lambda i: (i + NT, 0)),
            pl.BlockSpec((BLOCK_M, 2), lambda i: (i, 0)),
        ],
        out_specs=pl.BlockSpec((BLOCK_M, C), lambda i: (i, 0)),
        out_shape=jax.ShapeDtypeStruct((T, C), _F32),
    )(x2, g, g, rw)


def kernel(x, ln1_w, w_attn, w_proj, ln2_w, w_router, w1, w2):
    x2d = x.reshape(T, C)
    qkv = _k1(x2d, ln1_w.reshape(1, C), w_attn)
    y2d = _k2(qkv)
    x2, h2, lg = _k3(y2d, x2d, w_proj, ln2_w.reshape(1, C), w_router)
    ranks, rw, colr = _k4(lg)
    idx_all = ranks.T.reshape(R)
    col = colr[:, 0]
    xg = _sc_scatter(h2, idx_all)
    y2 = _k6(col, xg, w1, w2)
    g = _sc_gather(y2, idx_all)
    out = _k8(x2, g, rw)
    return out.reshape(B, T, C), lg
